# baseline (speedup 1.0000x reference)
import functools

import jax
import jax.numpy as jnp
from jax import lax
from jax.experimental import pallas as pl
from jax.experimental.pallas import tpu as pltpu
from jax.experimental.pallas import tpu_sc as plsc

_M = 48
_K = 256
_DSUB = 16
_B = 16384
_D = _M * _DSUB

_NC = 2
_NS = 16
_NW = _NC * _NS
_TOTAL = _B * _M
_PER_W = _TOTAL // _NW
_RPG = 128
_NG = _PER_W // _RPG
_KF = 8
_BURST = _KF * _RPG
_NB = _NG // _KF

_mesh = plsc.VectorSubcoreMesh(core_axis_name="c", subcore_axis_name="s")


@functools.partial(
    pl.kernel,
    mesh=_mesh,
    out_type=jax.ShapeDtypeStruct((_TOTAL, _DSUB), jnp.float32),
    scratch_types=[
        pltpu.VMEM((_NG, _RPG), jnp.int32),
        pltpu.VMEM((2, _BURST, _DSUB), jnp.float32),
        pltpu.SemaphoreType.DMA,
        pltpu.SemaphoreType.DMA,
    ],
    compiler_params=pltpu.CompilerParams(use_tc_tiling_on_sc=False),
)
def _pq_gather(codes_hbm, table_hbm, out_hbm, idx_v, rows_v, sem_g, sem_s):
    wid = lax.axis_index("s") * _NC + lax.axis_index("c")

    pltpu.sync_copy(codes_hbm.at[pl.ds(wid * _NG, _NG)], idx_v)

    lane = lax.iota(jnp.int32, 16)
    mmoff = lax.rem(lane, 8) * _K

    def add_offsets(j, carry):
        for o in range(_RPG // 16):
            c = lax.rem(2 * j + (o // 4), 6)
            off = c * (8 * _K) + mmoff
            sl = pl.ds(o * 16, 16)
            idx_v[j, sl] = idx_v[j, sl] + off
        return carry

    lax.fori_loop(0, _NG, add_offsets, 0)

    def burst_pair(i, carry):
        for b2 in range(2):
            g = 2 * i + b2

            @pl.when(g >= 2)
            def _drain_prev():
                pltpu.make_async_copy(
                    rows_v.at[b2],
                    out_hbm.at[pl.ds(wid * _PER_W, _BURST)],
                    sem_s,
                ).wait()

            copies = []
            for f in range(_KF):
                copies.append(
                    pltpu.async_copy(
                        table_hbm.at[idx_v.at[g * _KF + f]],
                        rows_v.at[b2, pl.ds(f * _RPG, _RPG)],
                        sem_g,
                    )
                )
            for c in copies:
                c.wait()
            pltpu.async_copy(
                rows_v.at[b2],
                out_hbm.at[pl.ds(wid * _PER_W + g * _BURST, _BURST)],
                sem_s,
            )
        return carry

    lax.fori_loop(0, _NB // 2, burst_pair, 0)

    for b2 in range(2):
        pltpu.make_async_copy(
            rows_v.at[b2],
            out_hbm.at[pl.ds(wid * _PER_W, _BURST)],
            sem_s,
        ).wait()


_RB = 64
_RG = _B // _RB


def _relayout_body(in_ref, out_ref):
    x = in_ref[...]
    for g in range(_RB // 8):
        for c in range(_D // 128):
            r = 48 * g + 8 * c
            out_ref[pl.ds(8 * g, 8), pl.ds(128 * c, 128)] = x[r:r + 8, :]


_relayout = pl.pallas_call(
    _relayout_body,
    grid=(_RG,),
    in_specs=[pl.BlockSpec((_RB * _D // 128, 128), lambda i: (i, 0))],
    out_specs=pl.BlockSpec((_RB, _D), lambda i: (i, 0)),
    out_shape=jax.ShapeDtypeStruct((_B, _D), jnp.float32),
)


def kernel(doc_codes, sub_weights):
    codes = (
        doc_codes.astype(jnp.int32)
        .reshape(_B // 8, 8, 6, 8)
        .transpose(0, 2, 1, 3)
        .reshape(_NW * _NG, _RPG)
    )
    table = sub_weights.reshape(_M * _K, _DSUB)
    flat = _pq_gather(codes, table)
    return _relayout(flat.reshape(_TOTAL * _DSUB // 128, 128))


# speedup vs baseline: 2.6174x; 2.6174x over previous
import functools

import jax
import jax.numpy as jnp
from jax import lax
from jax.experimental import pallas as pl
from jax.experimental.pallas import tpu as pltpu
from jax.experimental.pallas import tpu_sc as plsc

_M = 48
_K = 256
_DSUB = 16
_B = 16384
_D = _M * _DSUB

_NC = 2
_NS = 16
_NW = _NC * _NS
_TOTAL = _B * _M
_PER_W = _TOTAL // _NW
_RPG = 128
_NG = _PER_W // _RPG
_KF = 8
_BURST = _KF * _RPG
_NB = _NG // _KF

_mesh = plsc.VectorSubcoreMesh(core_axis_name="c", subcore_axis_name="s")


@functools.partial(
    pl.kernel,
    mesh=_mesh,
    out_type=jax.ShapeDtypeStruct((_TOTAL, _DSUB), jnp.float32),
    scratch_types=[
        pltpu.VMEM((_NG, _RPG), jnp.int32),
        pltpu.VMEM((3, _BURST, _DSUB), jnp.float32),
        pltpu.SemaphoreType.DMA,
        pltpu.SemaphoreType.DMA,
        pltpu.SemaphoreType.DMA,
    ],
    compiler_params=pltpu.CompilerParams(use_tc_tiling_on_sc=False),
)
def _pq_gather(codes_hbm, table_hbm, out_hbm, idx_v, rows_v, sem_ga, sem_gb,
               sem_s):
    wid = lax.axis_index("s") * _NC + lax.axis_index("c")

    pltpu.sync_copy(codes_hbm.at[pl.ds(wid * _NG, _NG)], idx_v)

    lane = lax.iota(jnp.int32, 16)

    def add_offsets(j, carry):
        for o in range(_RPG // 16):
            pos = j * _RPG + (o * 16) + lane
            off = lax.rem(pos, _M) * _K
            sl = pl.ds(o * 16, 16)
            idx_v[j, sl] = idx_v[j, sl] + off
        return carry

    lax.fori_loop(0, _NG, add_offsets, 0)

    def fire(g, buf, sem):
        for f in range(_KF):
            pltpu.async_copy(
                table_hbm.at[idx_v.at[g * _KF + f]],
                rows_v.at[buf, pl.ds(f * _RPG, _RPG)],
                sem,
            )

    def drain_gathers(buf, sem):
        pltpu.make_async_copy(
            out_hbm.at[pl.ds(0, _BURST)], rows_v.at[buf], sem
        ).wait()

    def scatter(g, buf):
        pltpu.async_copy(
            rows_v.at[buf],
            out_hbm.at[pl.ds(wid * _PER_W + g * _BURST, _BURST)],
            sem_s,
        )

    def drain_scatter(buf):
        pltpu.make_async_copy(
            rows_v.at[buf], out_hbm.at[pl.ds(wid * _PER_W, _BURST)], sem_s
        ).wait()

    sems = (sem_ga, sem_gb)

    def burst_pair(i, carry):
        for b2 in range(2):
            g = 2 * i + b2
            buf = lax.rem(g, 3)

            @pl.when(g >= 3)
            def _scatter_done():
                drain_scatter(buf)

            fire(g, buf, sems[b2])

            @pl.when(g >= 1)
            def _prev_done():
                prev = lax.rem(g + 2, 3)
                drain_gathers(prev, sems[1 - b2])
                scatter(g - 1, prev)

        return carry

    lax.fori_loop(0, _NB // 2, burst_pair, 0)

    last = lax.rem(_NB - 1, 3)
    drain_gathers(last, sems[(_NB - 1) % 2])
    scatter(_NB - 1, last)
    for _ in range(3):
        drain_scatter(0)


def kernel(doc_codes, sub_weights):
    codes = doc_codes.astype(jnp.int32).reshape(_NW * _NG, _RPG)
    table = sub_weights.reshape(_M * _K, _DSUB)
    flat = _pq_gather(codes, table)
    return flat.reshape(_B, _D)
